# Initial kernel scaffold; baseline (speedup 1.0000x reference)
import jax
import jax.numpy as jnp
from jax.experimental import pallas as pl


def kernel(x, edge_index, batch, W0, b0, W1, b1, W2, b2, a):
    raise NotImplementedError("write your pallas kernel here")


# speedup vs baseline: 13.7004x; 13.7004x over previous
import functools

import jax
import jax.numpy as jnp
from jax import lax
from jax.experimental import pallas as pl
from jax.experimental.pallas import tpu as pltpu
from jax.experimental.pallas import tpu_sc as plsc

N = 10000
E = 160000
D = 256
H = 128
G = 64
NC = 2
NS = 16
LANES = 16

EPT = E // NS
CH = 80
NCHUNK = EPT // CH
RP8 = 624
TAIL0 = NS * RP8
TAILN = N - TAIL0

EPW = E // (NC * NS)
EPW_PAD = ((EPW + 15) // 16) * 16

RB = 1000
NRB = N // RB

_MESH = plsc.VectorSubcoreMesh(core_axis_name="c", subcore_axis_name="s",
                               num_cores=NC, num_subcores=NS)


@functools.partial(
    pl.kernel,
    out_type=jax.ShapeDtypeStruct((NC * NS, N), jnp.float32),
    mesh=_MESH,
    compiler_params=pltpu.CompilerParams(needs_layout_passes=False),
    scratch_types=[
        pltpu.VMEM((EPW_PAD,), jnp.int32),
        pltpu.VMEM((N,), jnp.float32),
    ],
)
def _sc_degree(dst_hbm, out_hbm, dstv, hist):
    wid = lax.axis_index("s") * NC + lax.axis_index("c")
    pltpu.sync_copy(dst_hbm.at[wid], dstv)

    zeros16 = jnp.zeros((LANES,), jnp.float32)

    @pl.loop(0, N // LANES)
    def _zero(i):
        hist[pl.ds(i * LANES, LANES)] = zeros16

    ones16 = jnp.ones((LANES,), jnp.float32)

    @pl.loop(0, EPW // LANES)
    def _accum(i):
        idx = dstv[pl.ds(i * LANES, LANES)]
        plsc.addupdate_scatter(hist, [idx], ones16)

    tail = EPW - (EPW // LANES) * LANES
    if tail:
        idx = dstv[pl.ds((EPW // LANES) * LANES, LANES)]
        mask = lax.iota(jnp.int32, LANES) < tail
        plsc.addupdate_scatter(hist, [idx], ones16, mask=mask)

    pltpu.sync_copy(hist, out_hbm.at[wid])


@functools.partial(
    pl.kernel,
    out_type=jax.ShapeDtypeStruct((NC * N, H), jnp.float32),
    mesh=_MESH,
    compiler_params=pltpu.CompilerParams(needs_layout_passes=False),
    scratch_types=[
        pltpu.VMEM((NCHUNK, CH), jnp.int32),
        pltpu.VMEM((2, CH), jnp.int32),
        pltpu.VMEM((2, CH), jnp.int32),
        pltpu.VMEM((2, CH, H), jnp.float32),
        pltpu.VMEM_SHARED((N, H), jnp.float32),
        pltpu.SemaphoreType.DMA,
        pltpu.SemaphoreType.DMA,
    ],
)
def _sc_mp(zf_hbm, pk_hbm, out_hbm, pkv, srcw, dstw, gbuf, acc, s0, s1):
    c = lax.axis_index("c")
    s = lax.axis_index("s")
    row0 = s * RP8

    pltpu.sync_copy(zf_hbm.at[pl.ds(c * N + row0, RP8)],
                    acc.at[pl.ds(row0, RP8)])

    @pl.when(s == NS - 1)
    def _init_tail():
        pltpu.sync_copy(zf_hbm.at[pl.ds(c * N + TAIL0, TAILN)],
                        acc.at[pl.ds(TAIL0, TAILN)])

    pltpu.sync_copy(pk_hbm.at[s], pkv)

    plsc.subcore_barrier()

    offv = jnp.full((LANES,), c * N, jnp.int32)
    lo16 = jnp.full((LANES,), 0xFFFF, jnp.int32)

    def _unpack(j, buf):
        for k in range(CH // LANES):
            sl = pl.ds(k * LANES, LANES)
            pv = pkv[j, sl]
            srcw[buf, sl] = (pv & lo16) + offv
            dstw[buf, sl] = lax.shift_right_logical(pv, 16)

    def _gather(buf, sem):
        return pltpu.async_copy(zf_hbm.at[srcw.at[buf]], gbuf.at[buf], sem)

    def _scatter(buf):
        pltpu.sync_copy(gbuf.at[buf], acc.at[dstw.at[buf]], add=True)

    @pl.loop(0, NCHUNK // 2)
    def _pairs(i):
        j0 = 2 * i
        _unpack(j0, 0)
        g0 = _gather(0, s0)
        _unpack(j0 + 1, 1)
        g1 = _gather(1, s1)
        g0.wait()
        _scatter(0)
        g1.wait()
        _scatter(1)

    _unpack(NCHUNK - 1, 0)
    _gather(0, s0).wait()
    _scatter(0)

    plsc.subcore_barrier()
    pltpu.sync_copy(acc.at[pl.ds(row0, RP8)],
                    out_hbm.at[pl.ds(c * N + row0, RP8)])

    @pl.when(s == NS - 1)
    def _wb_tail():
        pltpu.sync_copy(acc.at[pl.ds(TAIL0, TAILN)],
                        out_hbm.at[pl.ds(c * N + TAIL0, TAILN)])


def _tc_prep_body(x_ref, w_ref, degp_ref, z_ref, dinv_ref):
    ones32 = jnp.ones((NC * NS, 1), jnp.float32)
    degp = degp_ref[...].reshape(NC * NS, RB)
    deg = lax.dot_general(degp, ones32,
                          (((0,), (0,)), ((), ())),
                          preferred_element_type=jnp.float32)
    dv = lax.rsqrt(deg + 1.0)
    dinv_ref[...] = dv
    xw = jnp.dot(x_ref[...], w_ref[...], preferred_element_type=jnp.float32)
    z = xw * dv
    z_ref[0] = z[:, :H]
    z_ref[1] = z[:, H:]


_tc_prep = pl.pallas_call(
    _tc_prep_body,
    grid=(NRB,),
    in_specs=[
        pl.BlockSpec((RB, D), lambda i: (i, 0)),
        pl.BlockSpec((D, D), lambda i: (0, 0)),
        pl.BlockSpec((1, NC * NS, RB), lambda i: (i, 0, 0)),
    ],
    out_specs=[
        pl.BlockSpec((2, RB, H), lambda i: (0, i, 0)),
        pl.BlockSpec((RB, 1), lambda i: (i, 0)),
    ],
    out_shape=[
        jax.ShapeDtypeStruct((2, N, H), jnp.float32),
        jax.ShapeDtypeStruct((N, 1), jnp.float32),
    ],
)


def _prelu(t, av):
    return jnp.where(t >= 0, t, av * t)


def _tc_layer_body(acc_ref, dinv_ref, w_ref, b_ref, a_ref, h_ref, z_ref):
    dv = dinv_ref[...]
    av = a_ref[0, 0]
    h0 = _prelu(acc_ref[0] * dv + b_ref[:, :H], av)
    h1 = _prelu(acc_ref[1] * dv + b_ref[:, H:], av)
    h = jnp.concatenate([h0, h1], axis=1)
    h_ref[...] = h
    xw = jnp.dot(h, w_ref[...], preferred_element_type=jnp.float32)
    z = xw * dv
    z_ref[0] = z[:, :H]
    z_ref[1] = z[:, H:]


_tc_layer = pl.pallas_call(
    _tc_layer_body,
    grid=(NRB,),
    in_specs=[
        pl.BlockSpec((2, RB, H), lambda i: (0, i, 0)),
        pl.BlockSpec((RB, 1), lambda i: (i, 0)),
        pl.BlockSpec((D, D), lambda i: (0, 0)),
        pl.BlockSpec((1, D), lambda i: (0, 0)),
        pl.BlockSpec(memory_space=pltpu.SMEM),
    ],
    out_specs=[
        pl.BlockSpec((RB, D), lambda i: (i, 0)),
        pl.BlockSpec((2, RB, H), lambda i: (0, i, 0)),
    ],
    out_shape=[
        jax.ShapeDtypeStruct((N, D), jnp.float32),
        jax.ShapeDtypeStruct((2, N, H), jnp.float32),
    ],
)


def _tc_final_body(acc_ref, dinv_ref, b_ref, a_ref, h1_ref, h2_ref, bat_ref,
                   h3_ref, pooled_ref):
    dv = dinv_ref[...]
    av = a_ref[0, 0]
    p0 = _prelu(acc_ref[0] * dv + b_ref[:, :H], av)
    p1 = _prelu(acc_ref[1] * dv + b_ref[:, H:], av)
    h3 = jnp.concatenate([p0, p1], axis=1)
    h3_ref[...] = h3

    @pl.when(pl.program_id(0) == 0)
    def _init():
        pooled_ref[...] = jnp.zeros((G, 3 * D), jnp.float32)

    bat = bat_ref[...].reshape(1, RB)
    gid = lax.broadcasted_iota(jnp.int32, (G, RB), 0)
    ind = jnp.where(gid == bat, 1.0, 0.0)
    pooled_ref[:, 0:D] += jnp.dot(ind, h1_ref[...],
                                  preferred_element_type=jnp.float32)
    pooled_ref[:, D:2 * D] += jnp.dot(ind, h2_ref[...],
                                      preferred_element_type=jnp.float32)
    pooled_ref[:, 2 * D:3 * D] += jnp.dot(ind, h3,
                                          preferred_element_type=jnp.float32)


_tc_final = pl.pallas_call(
    _tc_final_body,
    grid=(NRB,),
    in_specs=[
        pl.BlockSpec((2, RB, H), lambda i: (0, i, 0)),
        pl.BlockSpec((RB, 1), lambda i: (i, 0)),
        pl.BlockSpec((1, D), lambda i: (0, 0)),
        pl.BlockSpec(memory_space=pltpu.SMEM),
        pl.BlockSpec((RB, D), lambda i: (i, 0)),
        pl.BlockSpec((RB, D), lambda i: (i, 0)),
        pl.BlockSpec((1, 1, RB), lambda i: (i, 0, 0)),
    ],
    out_specs=[
        pl.BlockSpec((RB, D), lambda i: (i, 0)),
        pl.BlockSpec((G, 3 * D), lambda i: (0, 0)),
    ],
    out_shape=[
        jax.ShapeDtypeStruct((N, D), jnp.float32),
        jax.ShapeDtypeStruct((G, 3 * D), jnp.float32),
    ],
    compiler_params=pltpu.CompilerParams(
        dimension_semantics=("arbitrary",)),
)


def kernel(x, edge_index, batch, W0, b0, W1, b1, W2, b2, a):
    src = edge_index[0]
    dst = edge_index[1]

    dstd = jnp.concatenate(
        [dst.reshape(NC * NS, EPW),
         jnp.zeros((NC * NS, EPW_PAD - EPW), jnp.int32)], axis=1)
    pk = ((dst << 16) | src).reshape(NS, NCHUNK, CH)

    batr = batch.reshape(NRB, 1, RB)
    b0r = b0.reshape(1, D)
    b1r = b1.reshape(1, D)
    b2r = b2.reshape(1, D)
    ar = a.reshape(1, 1)

    degp = _sc_degree(dstd)
    degpt = jnp.transpose(degp.reshape(NC * NS, NRB, RB), (1, 0, 2))
    z0, dinv = _tc_prep(x, W0, degpt)

    acc0 = _sc_mp(z0.reshape(2 * N, H), pk)
    h1, z1 = _tc_layer(acc0.reshape(2, N, H), dinv, W1, b0r, ar)

    acc1 = _sc_mp(z1.reshape(2 * N, H), pk)
    h2, z2 = _tc_layer(acc1.reshape(2, N, H), dinv, W2, b1r, ar)

    acc2 = _sc_mp(z2.reshape(2 * N, H), pk)
    h3, pooled = _tc_final(acc2.reshape(2, N, H), dinv, b2r, ar, h1, h2, batr)

    return (pooled, h3)


# speedup vs baseline: 17.0467x; 1.2443x over previous
import functools

import jax
import jax.numpy as jnp
from jax import lax
from jax.experimental import pallas as pl
from jax.experimental.pallas import tpu as pltpu
from jax.experimental.pallas import tpu_sc as plsc

N = 10000
E = 160000
D = 256
H = 128
G = 64
NC = 2
NS = 16
LANES = 16

EPT = E // NS
CH = 80
NCHUNK = EPT // CH
RP8 = 624
TAIL0 = NS * RP8
TAILN = N - TAIL0

EPW = E // (NC * NS)
EPW_PAD = ((EPW + 15) // 16) * 16

RB = 1000
NRB = N // RB

_MESH = plsc.VectorSubcoreMesh(core_axis_name="c", subcore_axis_name="s",
                               num_cores=NC, num_subcores=NS)


@functools.partial(
    pl.kernel,
    out_type=jax.ShapeDtypeStruct((NC * NS, N), jnp.float32),
    mesh=_MESH,
    compiler_params=pltpu.CompilerParams(needs_layout_passes=False),
    scratch_types=[
        pltpu.VMEM((EPW_PAD,), jnp.int32),
        pltpu.VMEM((N,), jnp.float32),
    ],
)
def _sc_degree(dst_hbm, out_hbm, dstv, hist):
    wid = lax.axis_index("s") * NC + lax.axis_index("c")
    pltpu.sync_copy(dst_hbm.at[wid], dstv)

    zeros16 = jnp.zeros((LANES,), jnp.float32)

    @pl.loop(0, N // LANES)
    def _zero(i):
        hist[pl.ds(i * LANES, LANES)] = zeros16

    ones16 = jnp.ones((LANES,), jnp.float32)

    @pl.loop(0, EPW // LANES)
    def _accum(i):
        idx = dstv[pl.ds(i * LANES, LANES)]
        plsc.addupdate_scatter(hist, [idx], ones16)

    tail = EPW - (EPW // LANES) * LANES
    if tail:
        idx = dstv[pl.ds((EPW // LANES) * LANES, LANES)]
        mask = lax.iota(jnp.int32, LANES) < tail
        plsc.addupdate_scatter(hist, [idx], ones16, mask=mask)

    pltpu.sync_copy(hist, out_hbm.at[wid])


@functools.partial(
    pl.kernel,
    out_type=jax.ShapeDtypeStruct((NC * N, H), jnp.float32),
    mesh=_MESH,
    compiler_params=pltpu.CompilerParams(needs_layout_passes=False),
    scratch_types=[
        pltpu.VMEM((NCHUNK, CH), jnp.int32),
        pltpu.VMEM((2, CH), jnp.int32),
        pltpu.VMEM((2, CH), jnp.int32),
        pltpu.VMEM((2, CH, H), jnp.float32),
        pltpu.VMEM_SHARED((N, H), jnp.float32),
        pltpu.SemaphoreType.DMA,
        pltpu.SemaphoreType.DMA,
        pltpu.SemaphoreType.DMA,
        pltpu.SemaphoreType.DMA,
    ],
)
def _sc_mp(zf_hbm, pk_hbm, out_hbm, pkv, srcw, dstw, gbuf, acc,
           g0s, g1s, t0s, t1s):
    c = lax.axis_index("c")
    s = lax.axis_index("s")
    row0 = s * RP8

    pltpu.sync_copy(zf_hbm.at[pl.ds(c * N + row0, RP8)],
                    acc.at[pl.ds(row0, RP8)])

    @pl.when(s == NS - 1)
    def _init_tail():
        pltpu.sync_copy(zf_hbm.at[pl.ds(c * N + TAIL0, TAILN)],
                        acc.at[pl.ds(TAIL0, TAILN)])

    pltpu.sync_copy(pk_hbm.at[s], pkv)

    plsc.subcore_barrier()

    offv = jnp.full((LANES,), c * N, jnp.int32)
    lo16 = jnp.full((LANES,), 0xFFFF, jnp.int32)

    def _unpack(j, buf):
        for k in range(CH // LANES):
            sl = pl.ds(k * LANES, LANES)
            pv = pkv[j, sl]
            srcw[buf, sl] = (pv & lo16) + offv
            dstw[buf, sl] = lax.shift_right_logical(pv, 16)

    gsem = (g0s, g1s)
    ssem = (t0s, t1s)

    def _gather(buf):
        return pltpu.async_copy(zf_hbm.at[srcw.at[buf]], gbuf.at[buf],
                                gsem[buf])

    def _scatter(buf):
        return pltpu.async_copy(gbuf.at[buf], acc.at[dstw.at[buf]],
                                ssem[buf], add=True)

    U = 25

    @pl.loop(0, NCHUNK // U)
    def _grp(g):
        j0 = g * U
        gd = [None] * U
        sd = [None] * U
        _unpack(j0, 0)
        gd[0] = _gather(0)
        _unpack(j0 + 1, 1)
        gd[1] = _gather(1)
        gd[0].wait()
        sd[0] = _scatter(0)
        for u in range(2, U):
            b = u % 2
            sd[u - 2].wait()
            _unpack(j0 + u, b)
            gd[u] = _gather(b)
            gd[u - 1].wait()
            sd[u - 1] = _scatter(1 - b)
        gd[U - 1].wait()
        sd[U - 1] = _scatter((U - 1) % 2)
        sd[U - 2].wait()
        sd[U - 1].wait()

    plsc.subcore_barrier()
    pltpu.sync_copy(acc.at[pl.ds(row0, RP8)],
                    out_hbm.at[pl.ds(c * N + row0, RP8)])

    @pl.when(s == NS - 1)
    def _wb_tail():
        pltpu.sync_copy(acc.at[pl.ds(TAIL0, TAILN)],
                        out_hbm.at[pl.ds(c * N + TAIL0, TAILN)])


def _tc_prep_body(x_ref, w_ref, degp_ref, z_ref, dinv_ref):
    ones32 = jnp.ones((NC * NS, 1), jnp.float32)
    degp = degp_ref[...].reshape(NC * NS, RB)
    deg = lax.dot_general(degp, ones32,
                          (((0,), (0,)), ((), ())),
                          preferred_element_type=jnp.float32)
    dv = lax.rsqrt(deg + 1.0)
    dinv_ref[...] = dv
    xw = jnp.dot(x_ref[...], w_ref[...], preferred_element_type=jnp.float32)
    z = xw * dv
    z_ref[0] = z[:, :H]
    z_ref[1] = z[:, H:]


_tc_prep = pl.pallas_call(
    _tc_prep_body,
    grid=(NRB,),
    in_specs=[
        pl.BlockSpec((RB, D), lambda i: (i, 0)),
        pl.BlockSpec((D, D), lambda i: (0, 0)),
        pl.BlockSpec((1, NC * NS, RB), lambda i: (i, 0, 0)),
    ],
    out_specs=[
        pl.BlockSpec((2, RB, H), lambda i: (0, i, 0)),
        pl.BlockSpec((RB, 1), lambda i: (i, 0)),
    ],
    out_shape=[
        jax.ShapeDtypeStruct((2, N, H), jnp.float32),
        jax.ShapeDtypeStruct((N, 1), jnp.float32),
    ],
)


def _prelu(t, av):
    return jnp.where(t >= 0, t, av * t)


def _tc_layer_body(acc_ref, dinv_ref, w_ref, b_ref, a_ref, h_ref, z_ref):
    dv = dinv_ref[...]
    av = a_ref[0, 0]
    h0 = _prelu(acc_ref[0] * dv + b_ref[:, :H], av)
    h1 = _prelu(acc_ref[1] * dv + b_ref[:, H:], av)
    h = jnp.concatenate([h0, h1], axis=1)
    h_ref[...] = h
    xw = jnp.dot(h, w_ref[...], preferred_element_type=jnp.float32)
    z = xw * dv
    z_ref[0] = z[:, :H]
    z_ref[1] = z[:, H:]


_tc_layer = pl.pallas_call(
    _tc_layer_body,
    grid=(NRB,),
    in_specs=[
        pl.BlockSpec((2, RB, H), lambda i: (0, i, 0)),
        pl.BlockSpec((RB, 1), lambda i: (i, 0)),
        pl.BlockSpec((D, D), lambda i: (0, 0)),
        pl.BlockSpec((1, D), lambda i: (0, 0)),
        pl.BlockSpec(memory_space=pltpu.SMEM),
    ],
    out_specs=[
        pl.BlockSpec((RB, D), lambda i: (i, 0)),
        pl.BlockSpec((2, RB, H), lambda i: (0, i, 0)),
    ],
    out_shape=[
        jax.ShapeDtypeStruct((N, D), jnp.float32),
        jax.ShapeDtypeStruct((2, N, H), jnp.float32),
    ],
)


def _tc_final_body(acc_ref, dinv_ref, b_ref, a_ref, h1_ref, h2_ref, bat_ref,
                   h3_ref, pooled_ref):
    dv = dinv_ref[...]
    av = a_ref[0, 0]
    p0 = _prelu(acc_ref[0] * dv + b_ref[:, :H], av)
    p1 = _prelu(acc_ref[1] * dv + b_ref[:, H:], av)
    h3 = jnp.concatenate([p0, p1], axis=1)
    h3_ref[...] = h3

    @pl.when(pl.program_id(0) == 0)
    def _init():
        pooled_ref[...] = jnp.zeros((G, 3 * D), jnp.float32)

    bat = bat_ref[...].reshape(1, RB)
    gid = lax.broadcasted_iota(jnp.int32, (G, RB), 0)
    ind = jnp.where(gid == bat, 1.0, 0.0)
    pooled_ref[:, 0:D] += jnp.dot(ind, h1_ref[...],
                                  preferred_element_type=jnp.float32)
    pooled_ref[:, D:2 * D] += jnp.dot(ind, h2_ref[...],
                                      preferred_element_type=jnp.float32)
    pooled_ref[:, 2 * D:3 * D] += jnp.dot(ind, h3,
                                          preferred_element_type=jnp.float32)


_tc_final = pl.pallas_call(
    _tc_final_body,
    grid=(NRB,),
    in_specs=[
        pl.BlockSpec((2, RB, H), lambda i: (0, i, 0)),
        pl.BlockSpec((RB, 1), lambda i: (i, 0)),
        pl.BlockSpec((1, D), lambda i: (0, 0)),
        pl.BlockSpec(memory_space=pltpu.SMEM),
        pl.BlockSpec((RB, D), lambda i: (i, 0)),
        pl.BlockSpec((RB, D), lambda i: (i, 0)),
        pl.BlockSpec((1, 1, RB), lambda i: (i, 0, 0)),
    ],
    out_specs=[
        pl.BlockSpec((RB, D), lambda i: (i, 0)),
        pl.BlockSpec((G, 3 * D), lambda i: (0, 0)),
    ],
    out_shape=[
        jax.ShapeDtypeStruct((N, D), jnp.float32),
        jax.ShapeDtypeStruct((G, 3 * D), jnp.float32),
    ],
    compiler_params=pltpu.CompilerParams(
        dimension_semantics=("arbitrary",)),
)


def kernel(x, edge_index, batch, W0, b0, W1, b1, W2, b2, a):
    src = edge_index[0]
    dst = edge_index[1]

    dstd = jnp.concatenate(
        [dst.reshape(NC * NS, EPW),
         jnp.zeros((NC * NS, EPW_PAD - EPW), jnp.int32)], axis=1)
    pk = ((dst << 16) | src).reshape(NS, NCHUNK, CH)

    batr = batch.reshape(NRB, 1, RB)
    b0r = b0.reshape(1, D)
    b1r = b1.reshape(1, D)
    b2r = b2.reshape(1, D)
    ar = a.reshape(1, 1)

    degp = _sc_degree(dstd)
    degpt = jnp.transpose(degp.reshape(NC * NS, NRB, RB), (1, 0, 2))
    z0, dinv = _tc_prep(x, W0, degpt)

    acc0 = _sc_mp(z0.reshape(2 * N, H), pk)
    h1, z1 = _tc_layer(acc0.reshape(2, N, H), dinv, W1, b0r, ar)

    acc1 = _sc_mp(z1.reshape(2 * N, H), pk)
    h2, z2 = _tc_layer(acc1.reshape(2, N, H), dinv, W2, b1r, ar)

    acc2 = _sc_mp(z2.reshape(2 * N, H), pk)
    h3, pooled = _tc_final(acc2.reshape(2, N, H), dinv, b2r, ar, h1, h2, batr)

    return (pooled, h3)


# speedup vs baseline: 17.1932x; 1.0086x over previous
import functools

import jax
import jax.numpy as jnp
from jax import lax
from jax.experimental import pallas as pl
from jax.experimental.pallas import tpu as pltpu
from jax.experimental.pallas import tpu_sc as plsc

N = 10000
E = 160000
D = 256
H = 128
G = 64
NC = 2
NS = 16
LANES = 16

EPT = E // NS
CH = 80
NCHUNK = EPT // CH
RP8 = 624
TAIL0 = NS * RP8
TAILN = N - TAIL0

EPW = E // (NC * NS)
EPW_PAD = ((EPW + 15) // 16) * 16

RB = 1000
NRB = N // RB

_MESH = plsc.VectorSubcoreMesh(core_axis_name="c", subcore_axis_name="s",
                               num_cores=NC, num_subcores=NS)


@functools.partial(
    pl.kernel,
    out_type=jax.ShapeDtypeStruct((NC * NS, N), jnp.float32),
    mesh=_MESH,
    compiler_params=pltpu.CompilerParams(needs_layout_passes=False),
    scratch_types=[
        pltpu.VMEM((EPW_PAD,), jnp.int32),
        pltpu.VMEM((N,), jnp.float32),
    ],
)
def _sc_degree(dst_hbm, out_hbm, dstv, hist):
    wid = lax.axis_index("s") * NC + lax.axis_index("c")
    pltpu.sync_copy(dst_hbm.at[wid], dstv)

    zeros16 = jnp.zeros((LANES,), jnp.float32)

    @pl.loop(0, N // LANES)
    def _zero(i):
        hist[pl.ds(i * LANES, LANES)] = zeros16

    ones16 = jnp.ones((LANES,), jnp.float32)

    @pl.loop(0, EPW // LANES)
    def _accum(i):
        idx = dstv[pl.ds(i * LANES, LANES)]
        plsc.addupdate_scatter(hist, [idx], ones16)

    tail = EPW - (EPW // LANES) * LANES
    if tail:
        idx = dstv[pl.ds((EPW // LANES) * LANES, LANES)]
        mask = lax.iota(jnp.int32, LANES) < tail
        plsc.addupdate_scatter(hist, [idx], ones16, mask=mask)

    pltpu.sync_copy(hist, out_hbm.at[wid])


@functools.partial(
    pl.kernel,
    out_type=jax.ShapeDtypeStruct((NC * N, H), jnp.float32),
    mesh=_MESH,
    compiler_params=pltpu.CompilerParams(needs_layout_passes=False),
    scratch_types=[
        pltpu.VMEM((NCHUNK, CH), jnp.int32),
        pltpu.VMEM((2, CH), jnp.int32),
        pltpu.VMEM((2, CH), jnp.int32),
        pltpu.VMEM((2, CH, H), jnp.float32),
        pltpu.VMEM_SHARED((N, H), jnp.float32),
        pltpu.SemaphoreType.DMA,
        pltpu.SemaphoreType.DMA,
        pltpu.SemaphoreType.DMA,
        pltpu.SemaphoreType.DMA,
    ],
)
def _sc_mp(zf_hbm, pk_hbm, out_hbm, pkv, srcw, dstw, gbuf, acc,
           g0s, g1s, t0s, t1s):
    c = lax.axis_index("c")
    s = lax.axis_index("s")
    row0 = s * RP8

    pltpu.sync_copy(zf_hbm.at[pl.ds(c * N + row0, RP8)],
                    acc.at[pl.ds(row0, RP8)])

    @pl.when(s == NS - 1)
    def _init_tail():
        pltpu.sync_copy(zf_hbm.at[pl.ds(c * N + TAIL0, TAILN)],
                        acc.at[pl.ds(TAIL0, TAILN)])

    pltpu.sync_copy(pk_hbm.at[s], pkv)

    plsc.subcore_barrier()

    offv = jnp.full((LANES,), c * N, jnp.int32)
    lo16 = jnp.full((LANES,), 0xFFFF, jnp.int32)

    def _unpack(j, buf):
        for k in range(CH // LANES):
            sl = pl.ds(k * LANES, LANES)
            pv = pkv[j, sl]
            srcw[buf, sl] = (pv & lo16) + offv
            dstw[buf, sl] = lax.shift_right_logical(pv, 16)

    gsem = (g0s, g1s)
    ssem = (t0s, t1s)

    def _gather(buf):
        return pltpu.async_copy(zf_hbm.at[srcw.at[buf]], gbuf.at[buf],
                                gsem[buf])

    def _scatter(buf):
        return pltpu.async_copy(gbuf.at[buf], acc.at[dstw.at[buf]],
                                ssem[buf], add=True)

    U = 25

    @pl.loop(0, NCHUNK // U)
    def _grp(g):
        j0 = g * U
        gd = [None] * U
        sd = [None] * U
        _unpack(j0, 0)
        gd[0] = _gather(0)
        _unpack(j0 + 1, 1)
        gd[1] = _gather(1)
        gd[0].wait()
        sd[0] = _scatter(0)
        for u in range(2, U):
            b = u % 2
            sd[u - 2].wait()
            _unpack(j0 + u, b)
            gd[u] = _gather(b)
            gd[u - 1].wait()
            sd[u - 1] = _scatter(1 - b)
        gd[U - 1].wait()
        sd[U - 1] = _scatter((U - 1) % 2)
        sd[U - 2].wait()
        sd[U - 1].wait()

    plsc.subcore_barrier()
    pltpu.sync_copy(acc.at[pl.ds(row0, RP8)],
                    out_hbm.at[pl.ds(c * N + row0, RP8)])

    @pl.when(s == NS - 1)
    def _wb_tail():
        pltpu.sync_copy(acc.at[pl.ds(TAIL0, TAILN)],
                        out_hbm.at[pl.ds(c * N + TAIL0, TAILN)])


def _tc_prep_body(x_ref, w_ref, degp_ref, z_ref, dinv_ref):
    ones32 = jnp.ones((NC * NS, 1), jnp.float32)
    degp = degp_ref[...].reshape(NC * NS, RB)
    deg = lax.dot_general(degp, ones32,
                          (((0,), (0,)), ((), ())),
                          preferred_element_type=jnp.float32)
    dv = lax.rsqrt(deg + 1.0)
    dinv_ref[...] = dv
    xw = jnp.dot(x_ref[...], w_ref[...], preferred_element_type=jnp.float32)
    z = xw * dv
    z_ref[0] = z[:, :H]
    z_ref[1] = z[:, H:]


_tc_prep = pl.pallas_call(
    _tc_prep_body,
    grid=(NRB,),
    in_specs=[
        pl.BlockSpec((RB, D), lambda i: (i, 0)),
        pl.BlockSpec((D, D), lambda i: (0, 0)),
        pl.BlockSpec((1, NC * NS, RB), lambda i: (i, 0, 0)),
    ],
    out_specs=[
        pl.BlockSpec((2, RB, H), lambda i: (0, i, 0)),
        pl.BlockSpec((RB, 1), lambda i: (i, 0)),
    ],
    out_shape=[
        jax.ShapeDtypeStruct((2, N, H), jnp.float32),
        jax.ShapeDtypeStruct((N, 1), jnp.float32),
    ],
)


def _prelu(t, av):
    return jnp.where(t >= 0, t, av * t)


def _tc_layer_body(acc_ref, dinv_ref, w_ref, b_ref, a_ref, bat_ref,
                   z_ref, pool_ref):
    dv = dinv_ref[...]
    av = a_ref[0, 0]
    h0 = _prelu(acc_ref[0] * dv + b_ref[:, :H], av)
    h1 = _prelu(acc_ref[1] * dv + b_ref[:, H:], av)
    h = jnp.concatenate([h0, h1], axis=1)
    xw = jnp.dot(h, w_ref[...], preferred_element_type=jnp.float32)
    z = xw * dv
    z_ref[0] = z[:, :H]
    z_ref[1] = z[:, H:]

    @pl.when(pl.program_id(0) == 0)
    def _init():
        pool_ref[...] = jnp.zeros((G, D), jnp.float32)

    bat = bat_ref[...].reshape(1, RB)
    gid = lax.broadcasted_iota(jnp.int32, (G, RB), 0)
    ind = jnp.where(gid == bat, 1.0, 0.0)
    pool_ref[...] += jnp.dot(ind, h, preferred_element_type=jnp.float32)


_tc_layer = pl.pallas_call(
    _tc_layer_body,
    grid=(NRB,),
    in_specs=[
        pl.BlockSpec((2, RB, H), lambda i: (0, i, 0)),
        pl.BlockSpec((RB, 1), lambda i: (i, 0)),
        pl.BlockSpec((D, D), lambda i: (0, 0)),
        pl.BlockSpec((1, D), lambda i: (0, 0)),
        pl.BlockSpec(memory_space=pltpu.SMEM),
        pl.BlockSpec((1, 1, RB), lambda i: (i, 0, 0)),
    ],
    out_specs=[
        pl.BlockSpec((2, RB, H), lambda i: (0, i, 0)),
        pl.BlockSpec((G, D), lambda i: (0, 0)),
    ],
    out_shape=[
        jax.ShapeDtypeStruct((2, N, H), jnp.float32),
        jax.ShapeDtypeStruct((G, D), jnp.float32),
    ],
    compiler_params=pltpu.CompilerParams(
        dimension_semantics=("arbitrary",)),
)


def _tc_final_body(acc_ref, dinv_ref, b_ref, a_ref, bat_ref,
                   h3_ref, pool_ref):
    dv = dinv_ref[...]
    av = a_ref[0, 0]
    p0 = _prelu(acc_ref[0] * dv + b_ref[:, :H], av)
    p1 = _prelu(acc_ref[1] * dv + b_ref[:, H:], av)
    h3 = jnp.concatenate([p0, p1], axis=1)
    h3_ref[...] = h3

    @pl.when(pl.program_id(0) == 0)
    def _init():
        pool_ref[...] = jnp.zeros((G, D), jnp.float32)

    bat = bat_ref[...].reshape(1, RB)
    gid = lax.broadcasted_iota(jnp.int32, (G, RB), 0)
    ind = jnp.where(gid == bat, 1.0, 0.0)
    pool_ref[...] += jnp.dot(ind, h3, preferred_element_type=jnp.float32)


_tc_final = pl.pallas_call(
    _tc_final_body,
    grid=(NRB,),
    in_specs=[
        pl.BlockSpec((2, RB, H), lambda i: (0, i, 0)),
        pl.BlockSpec((RB, 1), lambda i: (i, 0)),
        pl.BlockSpec((1, D), lambda i: (0, 0)),
        pl.BlockSpec(memory_space=pltpu.SMEM),
        pl.BlockSpec((1, 1, RB), lambda i: (i, 0, 0)),
    ],
    out_specs=[
        pl.BlockSpec((RB, D), lambda i: (i, 0)),
        pl.BlockSpec((G, D), lambda i: (0, 0)),
    ],
    out_shape=[
        jax.ShapeDtypeStruct((N, D), jnp.float32),
        jax.ShapeDtypeStruct((G, D), jnp.float32),
    ],
    compiler_params=pltpu.CompilerParams(
        dimension_semantics=("arbitrary",)),
)


def kernel(x, edge_index, batch, W0, b0, W1, b1, W2, b2, a):
    src = edge_index[0]
    dst = edge_index[1]

    dstd = jnp.concatenate(
        [dst.reshape(NC * NS, EPW),
         jnp.zeros((NC * NS, EPW_PAD - EPW), jnp.int32)], axis=1)
    pk = ((dst << 16) | src).reshape(NS, NCHUNK, CH)

    batr = batch.reshape(NRB, 1, RB)
    b0r = b0.reshape(1, D)
    b1r = b1.reshape(1, D)
    b2r = b2.reshape(1, D)
    ar = a.reshape(1, 1)

    degp = _sc_degree(dstd)
    degpt = jnp.transpose(degp.reshape(NC * NS, NRB, RB), (1, 0, 2))
    z0, dinv = _tc_prep(x, W0, degpt)

    acc0 = _sc_mp(z0.reshape(2 * N, H), pk)
    z1, p1 = _tc_layer(acc0.reshape(2, N, H), dinv, W1, b0r, ar, batr)

    acc1 = _sc_mp(z1.reshape(2 * N, H), pk)
    z2, p2 = _tc_layer(acc1.reshape(2, N, H), dinv, W2, b1r, ar, batr)

    acc2 = _sc_mp(z2.reshape(2 * N, H), pk)
    h3, p3 = _tc_final(acc2.reshape(2, N, H), dinv, b2r, ar, batr)

    pooled = jnp.concatenate([p1, p2, p3], axis=1)
    return (pooled, h3)


# speedup vs baseline: 18.0320x; 1.0488x over previous
import functools

import jax
import jax.numpy as jnp
from jax import lax
from jax.experimental import pallas as pl
from jax.experimental.pallas import tpu as pltpu
from jax.experimental.pallas import tpu_sc as plsc

N = 10000
E = 160000
D = 256
H = 128
G = 64
NC = 2
NS = 16
LANES = 16

EPT = E // NS
CH = 100
NCHUNK = EPT // CH
RP8 = 624
TAIL0 = NS * RP8
TAILN = N - TAIL0

EPW = E // (NC * NS)
EPW_PAD = ((EPW + 15) // 16) * 16

RB = 1000
NRB = N // RB

_MESH = plsc.VectorSubcoreMesh(core_axis_name="c", subcore_axis_name="s",
                               num_cores=NC, num_subcores=NS)


@functools.partial(
    pl.kernel,
    out_type=jax.ShapeDtypeStruct((NC * NS, N), jnp.float32),
    mesh=_MESH,
    compiler_params=pltpu.CompilerParams(needs_layout_passes=False),
    scratch_types=[
        pltpu.VMEM((EPW_PAD,), jnp.int32),
        pltpu.VMEM((N,), jnp.float32),
    ],
)
def _sc_degree(dst_hbm, out_hbm, dstv, hist):
    wid = lax.axis_index("s") * NC + lax.axis_index("c")
    pltpu.sync_copy(dst_hbm.at[wid], dstv)

    zeros16 = jnp.zeros((LANES,), jnp.float32)

    @pl.loop(0, N // LANES)
    def _zero(i):
        hist[pl.ds(i * LANES, LANES)] = zeros16

    ones16 = jnp.ones((LANES,), jnp.float32)

    @pl.loop(0, EPW // LANES)
    def _accum(i):
        idx = dstv[pl.ds(i * LANES, LANES)]
        plsc.addupdate_scatter(hist, [idx], ones16)

    tail = EPW - (EPW // LANES) * LANES
    if tail:
        idx = dstv[pl.ds((EPW // LANES) * LANES, LANES)]
        mask = lax.iota(jnp.int32, LANES) < tail
        plsc.addupdate_scatter(hist, [idx], ones16, mask=mask)

    pltpu.sync_copy(hist, out_hbm.at[wid])


@functools.partial(
    pl.kernel,
    out_type=jax.ShapeDtypeStruct((NC * N, H), jnp.float32),
    mesh=_MESH,
    compiler_params=pltpu.CompilerParams(needs_layout_passes=False),
    scratch_types=[
        pltpu.VMEM((NCHUNK, CH), jnp.int32),
        pltpu.VMEM((2, CH), jnp.int32),
        pltpu.VMEM((2, CH), jnp.int32),
        pltpu.VMEM((2, CH, H), jnp.float32),
        pltpu.VMEM_SHARED((N, H), jnp.float32),
        pltpu.SemaphoreType.DMA,
        pltpu.SemaphoreType.DMA,
        pltpu.SemaphoreType.DMA,
        pltpu.SemaphoreType.DMA,
    ],
)
def _sc_mp(zf_hbm, pk_hbm, out_hbm, pkv, srcw, dstw, gbuf, acc,
           g0s, g1s, t0s, t1s):
    c = lax.axis_index("c")
    s = lax.axis_index("s")
    row0 = s * RP8

    pltpu.sync_copy(zf_hbm.at[pl.ds(c * N + row0, RP8)],
                    acc.at[pl.ds(row0, RP8)])

    @pl.when(s == NS - 1)
    def _init_tail():
        pltpu.sync_copy(zf_hbm.at[pl.ds(c * N + TAIL0, TAILN)],
                        acc.at[pl.ds(TAIL0, TAILN)])

    pltpu.sync_copy(pk_hbm.at[s], pkv)

    plsc.subcore_barrier()

    offv = jnp.full((LANES,), c * N, jnp.int32)
    lo16 = jnp.full((LANES,), 0xFFFF, jnp.int32)

    _STEPS = [k * LANES for k in range(CH // LANES)]
    if CH % LANES:
        _STEPS.append(CH - LANES)

    def _unpack(j, buf):
        for st in _STEPS:
            sl = pl.ds(st, LANES)
            pv = pkv[j, sl]
            srcw[buf, sl] = (pv & lo16) + offv
            dstw[buf, sl] = lax.shift_right_logical(pv, 16)

    gsem = (g0s, g1s)
    ssem = (t0s, t1s)

    def _gather(buf):
        return pltpu.async_copy(zf_hbm.at[srcw.at[buf]], gbuf.at[buf],
                                gsem[buf])

    def _scatter(buf):
        return pltpu.async_copy(gbuf.at[buf], acc.at[dstw.at[buf]],
                                ssem[buf], add=True)

    U = 25

    @pl.loop(0, NCHUNK // U)
    def _grp(g):
        j0 = g * U
        gd = [None] * U
        sd = [None] * U
        _unpack(j0, 0)
        gd[0] = _gather(0)
        _unpack(j0 + 1, 1)
        gd[1] = _gather(1)
        gd[0].wait()
        sd[0] = _scatter(0)
        for u in range(2, U):
            b = u % 2
            sd[u - 2].wait()
            _unpack(j0 + u, b)
            gd[u] = _gather(b)
            gd[u - 1].wait()
            sd[u - 1] = _scatter(1 - b)
        gd[U - 1].wait()
        sd[U - 1] = _scatter((U - 1) % 2)
        sd[U - 2].wait()
        sd[U - 1].wait()

    plsc.subcore_barrier()
    pltpu.sync_copy(acc.at[pl.ds(row0, RP8)],
                    out_hbm.at[pl.ds(c * N + row0, RP8)])

    @pl.when(s == NS - 1)
    def _wb_tail():
        pltpu.sync_copy(acc.at[pl.ds(TAIL0, TAILN)],
                        out_hbm.at[pl.ds(c * N + TAIL0, TAILN)])


def _tc_prep_body(x_ref, w_ref, degp_ref, z_ref, dinv_ref):
    ones32 = jnp.ones((NC * NS, 1), jnp.float32)
    degp = degp_ref[...].reshape(NC * NS, RB)
    deg = lax.dot_general(degp, ones32,
                          (((0,), (0,)), ((), ())),
                          preferred_element_type=jnp.float32)
    dv = lax.rsqrt(deg + 1.0)
    dinv_ref[...] = dv
    xw = jnp.dot(x_ref[...], w_ref[...], preferred_element_type=jnp.float32)
    z = xw * dv
    z_ref[0] = z[:, :H]
    z_ref[1] = z[:, H:]


_tc_prep = pl.pallas_call(
    _tc_prep_body,
    grid=(NRB,),
    in_specs=[
        pl.BlockSpec((RB, D), lambda i: (i, 0)),
        pl.BlockSpec((D, D), lambda i: (0, 0)),
        pl.BlockSpec((1, NC * NS, RB), lambda i: (i, 0, 0)),
    ],
    out_specs=[
        pl.BlockSpec((2, RB, H), lambda i: (0, i, 0)),
        pl.BlockSpec((RB, 1), lambda i: (i, 0)),
    ],
    out_shape=[
        jax.ShapeDtypeStruct((2, N, H), jnp.float32),
        jax.ShapeDtypeStruct((N, 1), jnp.float32),
    ],
)


def _prelu(t, av):
    return jnp.where(t >= 0, t, av * t)


def _tc_layer_body(acc_ref, dinv_ref, w_ref, b_ref, a_ref, bat_ref,
                   z_ref, pool_ref):
    dv = dinv_ref[...]
    av = a_ref[0, 0]
    h0 = _prelu(acc_ref[0] * dv + b_ref[:, :H], av)
    h1 = _prelu(acc_ref[1] * dv + b_ref[:, H:], av)
    h = jnp.concatenate([h0, h1], axis=1)
    xw = jnp.dot(h, w_ref[...], preferred_element_type=jnp.float32)
    z = xw * dv
    z_ref[0] = z[:, :H]
    z_ref[1] = z[:, H:]

    @pl.when(pl.program_id(0) == 0)
    def _init():
        pool_ref[...] = jnp.zeros((G, D), jnp.float32)

    bat = bat_ref[...].reshape(1, RB)
    gid = lax.broadcasted_iota(jnp.int32, (G, RB), 0)
    ind = jnp.where(gid == bat, 1.0, 0.0)
    pool_ref[...] += jnp.dot(ind, h, preferred_element_type=jnp.float32)


_tc_layer = pl.pallas_call(
    _tc_layer_body,
    grid=(NRB,),
    in_specs=[
        pl.BlockSpec((2, RB, H), lambda i: (0, i, 0)),
        pl.BlockSpec((RB, 1), lambda i: (i, 0)),
        pl.BlockSpec((D, D), lambda i: (0, 0)),
        pl.BlockSpec((1, D), lambda i: (0, 0)),
        pl.BlockSpec(memory_space=pltpu.SMEM),
        pl.BlockSpec((1, 1, RB), lambda i: (i, 0, 0)),
    ],
    out_specs=[
        pl.BlockSpec((2, RB, H), lambda i: (0, i, 0)),
        pl.BlockSpec((G, D), lambda i: (0, 0)),
    ],
    out_shape=[
        jax.ShapeDtypeStruct((2, N, H), jnp.float32),
        jax.ShapeDtypeStruct((G, D), jnp.float32),
    ],
    compiler_params=pltpu.CompilerParams(
        dimension_semantics=("arbitrary",)),
)


def _tc_final_body(acc_ref, dinv_ref, b_ref, a_ref, bat_ref,
                   h3_ref, pool_ref):
    dv = dinv_ref[...]
    av = a_ref[0, 0]
    p0 = _prelu(acc_ref[0] * dv + b_ref[:, :H], av)
    p1 = _prelu(acc_ref[1] * dv + b_ref[:, H:], av)
    h3 = jnp.concatenate([p0, p1], axis=1)
    h3_ref[...] = h3

    @pl.when(pl.program_id(0) == 0)
    def _init():
        pool_ref[...] = jnp.zeros((G, D), jnp.float32)

    bat = bat_ref[...].reshape(1, RB)
    gid = lax.broadcasted_iota(jnp.int32, (G, RB), 0)
    ind = jnp.where(gid == bat, 1.0, 0.0)
    pool_ref[...] += jnp.dot(ind, h3, preferred_element_type=jnp.float32)


_tc_final = pl.pallas_call(
    _tc_final_body,
    grid=(NRB,),
    in_specs=[
        pl.BlockSpec((2, RB, H), lambda i: (0, i, 0)),
        pl.BlockSpec((RB, 1), lambda i: (i, 0)),
        pl.BlockSpec((1, D), lambda i: (0, 0)),
        pl.BlockSpec(memory_space=pltpu.SMEM),
        pl.BlockSpec((1, 1, RB), lambda i: (i, 0, 0)),
    ],
    out_specs=[
        pl.BlockSpec((RB, D), lambda i: (i, 0)),
        pl.BlockSpec((G, D), lambda i: (0, 0)),
    ],
    out_shape=[
        jax.ShapeDtypeStruct((N, D), jnp.float32),
        jax.ShapeDtypeStruct((G, D), jnp.float32),
    ],
    compiler_params=pltpu.CompilerParams(
        dimension_semantics=("arbitrary",)),
)


def kernel(x, edge_index, batch, W0, b0, W1, b1, W2, b2, a):
    src = edge_index[0]
    dst = edge_index[1]

    dstd = jnp.concatenate(
        [dst.reshape(NC * NS, EPW),
         jnp.zeros((NC * NS, EPW_PAD - EPW), jnp.int32)], axis=1)
    pk = ((dst << 16) | src).reshape(NS, NCHUNK, CH)

    batr = batch.reshape(NRB, 1, RB)
    b0r = b0.reshape(1, D)
    b1r = b1.reshape(1, D)
    b2r = b2.reshape(1, D)
    ar = a.reshape(1, 1)

    degp = _sc_degree(dstd)
    degpt = jnp.transpose(degp.reshape(NC * NS, NRB, RB), (1, 0, 2))
    z0, dinv = _tc_prep(x, W0, degpt)

    acc0 = _sc_mp(z0.reshape(2 * N, H), pk)
    z1, p1 = _tc_layer(acc0.reshape(2, N, H), dinv, W1, b0r, ar, batr)

    acc1 = _sc_mp(z1.reshape(2 * N, H), pk)
    z2, p2 = _tc_layer(acc1.reshape(2, N, H), dinv, W2, b1r, ar, batr)

    acc2 = _sc_mp(z2.reshape(2 * N, H), pk)
    h3, p3 = _tc_final(acc2.reshape(2, N, H), dinv, b2r, ar, batr)

    pooled = jnp.concatenate([p1, p2, p3], axis=1)
    return (pooled, h3)


# speedup vs baseline: 18.6492x; 1.0342x over previous
import functools

import jax
import jax.numpy as jnp
from jax import lax
from jax.experimental import pallas as pl
from jax.experimental.pallas import tpu as pltpu
from jax.experimental.pallas import tpu_sc as plsc

N = 10000
E = 160000
D = 256
H = 128
G = 64
NC = 2
NS = 16
LANES = 16

EPT = E // NS
CH = 125
NCHUNK = EPT // CH
RP8 = 624
TAIL0 = NS * RP8
TAILN = N - TAIL0

EPW = E // (NC * NS)
EPW_PAD = ((EPW + 15) // 16) * 16

RB = 1000
NRB = N // RB

_MESH = plsc.VectorSubcoreMesh(core_axis_name="c", subcore_axis_name="s",
                               num_cores=NC, num_subcores=NS)


@functools.partial(
    pl.kernel,
    out_type=jax.ShapeDtypeStruct((NC * NS, N), jnp.float32),
    mesh=_MESH,
    compiler_params=pltpu.CompilerParams(needs_layout_passes=False),
    scratch_types=[
        pltpu.VMEM((EPW_PAD,), jnp.int32),
        pltpu.VMEM((N,), jnp.float32),
    ],
)
def _sc_degree(dst_hbm, out_hbm, dstv, hist):
    wid = lax.axis_index("s") * NC + lax.axis_index("c")
    pltpu.sync_copy(dst_hbm.at[wid], dstv)

    zeros16 = jnp.zeros((LANES,), jnp.float32)

    @pl.loop(0, N // LANES)
    def _zero(i):
        hist[pl.ds(i * LANES, LANES)] = zeros16

    ones16 = jnp.ones((LANES,), jnp.float32)

    @pl.loop(0, EPW // LANES)
    def _accum(i):
        idx = dstv[pl.ds(i * LANES, LANES)]
        plsc.addupdate_scatter(hist, [idx], ones16)

    tail = EPW - (EPW // LANES) * LANES
    if tail:
        idx = dstv[pl.ds((EPW // LANES) * LANES, LANES)]
        mask = lax.iota(jnp.int32, LANES) < tail
        plsc.addupdate_scatter(hist, [idx], ones16, mask=mask)

    pltpu.sync_copy(hist, out_hbm.at[wid])


@functools.partial(
    pl.kernel,
    out_type=jax.ShapeDtypeStruct((NC * N, H), jnp.float32),
    mesh=_MESH,
    compiler_params=pltpu.CompilerParams(needs_layout_passes=False),
    scratch_types=[
        pltpu.VMEM((NCHUNK, CH), jnp.int32),
        pltpu.VMEM((2, CH), jnp.int32),
        pltpu.VMEM((2, CH), jnp.int32),
        pltpu.VMEM((2, CH, H), jnp.float32),
        pltpu.VMEM_SHARED((N, H), jnp.float32),
        pltpu.SemaphoreType.DMA,
        pltpu.SemaphoreType.DMA,
        pltpu.SemaphoreType.DMA,
        pltpu.SemaphoreType.DMA,
    ],
)
def _sc_mp(zf_hbm, pk_hbm, out_hbm, pkv, srcw, dstw, gbuf, acc,
           g0s, g1s, t0s, t1s):
    c = lax.axis_index("c")
    s = lax.axis_index("s")
    row0 = s * RP8

    pltpu.sync_copy(zf_hbm.at[pl.ds(c * N + row0, RP8)],
                    acc.at[pl.ds(row0, RP8)])

    @pl.when(s == NS - 1)
    def _init_tail():
        pltpu.sync_copy(zf_hbm.at[pl.ds(c * N + TAIL0, TAILN)],
                        acc.at[pl.ds(TAIL0, TAILN)])

    pltpu.sync_copy(pk_hbm.at[s], pkv)

    plsc.subcore_barrier()

    offv = jnp.full((LANES,), c * N, jnp.int32)
    lo16 = jnp.full((LANES,), 0xFFFF, jnp.int32)

    _STEPS = [k * LANES for k in range(CH // LANES)]
    if CH % LANES:
        _STEPS.append(CH - LANES)

    def _unpack(j, buf):
        for st in _STEPS:
            sl = pl.ds(st, LANES)
            pv = pkv[j, sl]
            srcw[buf, sl] = (pv & lo16) + offv
            dstw[buf, sl] = lax.shift_right_logical(pv, 16)

    gsem = (g0s, g1s)
    ssem = (t0s, t1s)

    def _gather(buf):
        return pltpu.async_copy(zf_hbm.at[srcw.at[buf]], gbuf.at[buf],
                                gsem[buf])

    def _scatter(buf):
        return pltpu.async_copy(gbuf.at[buf], acc.at[dstw.at[buf]],
                                ssem[buf], add=True)

    U = 20

    @pl.loop(0, NCHUNK // U)
    def _grp(g):
        j0 = g * U
        gd = [None] * U
        sd = [None] * U
        _unpack(j0, 0)
        gd[0] = _gather(0)
        _unpack(j0 + 1, 1)
        gd[1] = _gather(1)
        gd[0].wait()
        sd[0] = _scatter(0)
        for u in range(2, U):
            b = u % 2
            sd[u - 2].wait()
            _unpack(j0 + u, b)
            gd[u] = _gather(b)
            gd[u - 1].wait()
            sd[u - 1] = _scatter(1 - b)
        gd[U - 1].wait()
        sd[U - 1] = _scatter((U - 1) % 2)
        sd[U - 2].wait()
        sd[U - 1].wait()

    plsc.subcore_barrier()
    pltpu.sync_copy(acc.at[pl.ds(row0, RP8)],
                    out_hbm.at[pl.ds(c * N + row0, RP8)])

    @pl.when(s == NS - 1)
    def _wb_tail():
        pltpu.sync_copy(acc.at[pl.ds(TAIL0, TAILN)],
                        out_hbm.at[pl.ds(c * N + TAIL0, TAILN)])


def _tc_prep_body(x_ref, w_ref, degp_ref, z_ref, dinv_ref):
    ones32 = jnp.ones((NC * NS, 1), jnp.float32)
    degp = degp_ref[...].reshape(NC * NS, RB)
    deg = lax.dot_general(degp, ones32,
                          (((0,), (0,)), ((), ())),
                          preferred_element_type=jnp.float32)
    dv = lax.rsqrt(deg + 1.0)
    dinv_ref[...] = dv
    xw = jnp.dot(x_ref[...], w_ref[...], preferred_element_type=jnp.float32)
    z = xw * dv
    z_ref[0] = z[:, :H]
    z_ref[1] = z[:, H:]


_tc_prep = pl.pallas_call(
    _tc_prep_body,
    grid=(NRB,),
    in_specs=[
        pl.BlockSpec((RB, D), lambda i: (i, 0)),
        pl.BlockSpec((D, D), lambda i: (0, 0)),
        pl.BlockSpec((1, NC * NS, RB), lambda i: (i, 0, 0)),
    ],
    out_specs=[
        pl.BlockSpec((2, RB, H), lambda i: (0, i, 0)),
        pl.BlockSpec((RB, 1), lambda i: (i, 0)),
    ],
    out_shape=[
        jax.ShapeDtypeStruct((2, N, H), jnp.float32),
        jax.ShapeDtypeStruct((N, 1), jnp.float32),
    ],
)


def _prelu(t, av):
    return jnp.where(t >= 0, t, av * t)


def _tc_layer_body(acc_ref, dinv_ref, w_ref, b_ref, a_ref, bat_ref,
                   z_ref, pool_ref):
    dv = dinv_ref[...]
    av = a_ref[0, 0]
    h0 = _prelu(acc_ref[0] * dv + b_ref[:, :H], av)
    h1 = _prelu(acc_ref[1] * dv + b_ref[:, H:], av)
    h = jnp.concatenate([h0, h1], axis=1)
    xw = jnp.dot(h, w_ref[...], preferred_element_type=jnp.float32)
    z = xw * dv
    z_ref[0] = z[:, :H]
    z_ref[1] = z[:, H:]

    @pl.when(pl.program_id(0) == 0)
    def _init():
        pool_ref[...] = jnp.zeros((G, D), jnp.float32)

    bat = bat_ref[...].reshape(1, RB)
    gid = lax.broadcasted_iota(jnp.int32, (G, RB), 0)
    ind = jnp.where(gid == bat, 1.0, 0.0)
    pool_ref[...] += jnp.dot(ind, h, preferred_element_type=jnp.float32)


_tc_layer = pl.pallas_call(
    _tc_layer_body,
    grid=(NRB,),
    in_specs=[
        pl.BlockSpec((2, RB, H), lambda i: (0, i, 0)),
        pl.BlockSpec((RB, 1), lambda i: (i, 0)),
        pl.BlockSpec((D, D), lambda i: (0, 0)),
        pl.BlockSpec((1, D), lambda i: (0, 0)),
        pl.BlockSpec(memory_space=pltpu.SMEM),
        pl.BlockSpec((1, 1, RB), lambda i: (i, 0, 0)),
    ],
    out_specs=[
        pl.BlockSpec((2, RB, H), lambda i: (0, i, 0)),
        pl.BlockSpec((G, D), lambda i: (0, 0)),
    ],
    out_shape=[
        jax.ShapeDtypeStruct((2, N, H), jnp.float32),
        jax.ShapeDtypeStruct((G, D), jnp.float32),
    ],
    compiler_params=pltpu.CompilerParams(
        dimension_semantics=("arbitrary",)),
)


def _tc_final_body(acc_ref, dinv_ref, b_ref, a_ref, bat_ref,
                   h3_ref, pool_ref):
    dv = dinv_ref[...]
    av = a_ref[0, 0]
    p0 = _prelu(acc_ref[0] * dv + b_ref[:, :H], av)
    p1 = _prelu(acc_ref[1] * dv + b_ref[:, H:], av)
    h3 = jnp.concatenate([p0, p1], axis=1)
    h3_ref[...] = h3

    @pl.when(pl.program_id(0) == 0)
    def _init():
        pool_ref[...] = jnp.zeros((G, D), jnp.float32)

    bat = bat_ref[...].reshape(1, RB)
    gid = lax.broadcasted_iota(jnp.int32, (G, RB), 0)
    ind = jnp.where(gid == bat, 1.0, 0.0)
    pool_ref[...] += jnp.dot(ind, h3, preferred_element_type=jnp.float32)


_tc_final = pl.pallas_call(
    _tc_final_body,
    grid=(NRB,),
    in_specs=[
        pl.BlockSpec((2, RB, H), lambda i: (0, i, 0)),
        pl.BlockSpec((RB, 1), lambda i: (i, 0)),
        pl.BlockSpec((1, D), lambda i: (0, 0)),
        pl.BlockSpec(memory_space=pltpu.SMEM),
        pl.BlockSpec((1, 1, RB), lambda i: (i, 0, 0)),
    ],
    out_specs=[
        pl.BlockSpec((RB, D), lambda i: (i, 0)),
        pl.BlockSpec((G, D), lambda i: (0, 0)),
    ],
    out_shape=[
        jax.ShapeDtypeStruct((N, D), jnp.float32),
        jax.ShapeDtypeStruct((G, D), jnp.float32),
    ],
    compiler_params=pltpu.CompilerParams(
        dimension_semantics=("arbitrary",)),
)


def kernel(x, edge_index, batch, W0, b0, W1, b1, W2, b2, a):
    src = edge_index[0]
    dst = edge_index[1]

    dstd = jnp.concatenate(
        [dst.reshape(NC * NS, EPW),
         jnp.zeros((NC * NS, EPW_PAD - EPW), jnp.int32)], axis=1)
    pk = ((dst << 16) | src).reshape(NS, NCHUNK, CH)

    batr = batch.reshape(NRB, 1, RB)
    b0r = b0.reshape(1, D)
    b1r = b1.reshape(1, D)
    b2r = b2.reshape(1, D)
    ar = a.reshape(1, 1)

    degp = _sc_degree(dstd)
    degpt = jnp.transpose(degp.reshape(NC * NS, NRB, RB), (1, 0, 2))
    z0, dinv = _tc_prep(x, W0, degpt)

    acc0 = _sc_mp(z0.reshape(2 * N, H), pk)
    z1, p1 = _tc_layer(acc0.reshape(2, N, H), dinv, W1, b0r, ar, batr)

    acc1 = _sc_mp(z1.reshape(2 * N, H), pk)
    z2, p2 = _tc_layer(acc1.reshape(2, N, H), dinv, W2, b1r, ar, batr)

    acc2 = _sc_mp(z2.reshape(2 * N, H), pk)
    h3, p3 = _tc_final(acc2.reshape(2, N, H), dinv, b2r, ar, batr)

    pooled = jnp.concatenate([p1, p2, p3], axis=1)
    return (pooled, h3)


# speedup vs baseline: 18.7404x; 1.0049x over previous
import functools

import jax
import jax.numpy as jnp
from jax import lax
from jax.experimental import pallas as pl
from jax.experimental.pallas import tpu as pltpu
from jax.experimental.pallas import tpu_sc as plsc

N = 10000
E = 160000
D = 256
H = 128
G = 64
NC = 2
NS = 16
LANES = 16

EPT = E // NS
CH = 125
NCHUNK = EPT // CH
RP8 = 624
TAIL0 = NS * RP8
TAILN = N - TAIL0

EPW = E // (NC * NS)
EPW_PAD = ((EPW + 15) // 16) * 16

RB = 1000
NRB = N // RB

_MESH = plsc.VectorSubcoreMesh(core_axis_name="c", subcore_axis_name="s",
                               num_cores=NC, num_subcores=NS)


@functools.partial(
    pl.kernel,
    out_type=jax.ShapeDtypeStruct((NC * NS, N), jnp.float32),
    mesh=_MESH,
    compiler_params=pltpu.CompilerParams(needs_layout_passes=False),
    scratch_types=[
        pltpu.VMEM((EPW_PAD,), jnp.int32),
        pltpu.VMEM((N,), jnp.float32),
    ],
)
def _sc_degree(dst_hbm, out_hbm, dstv, hist):
    wid = lax.axis_index("s") * NC + lax.axis_index("c")
    pltpu.sync_copy(dst_hbm.at[wid], dstv)

    zeros16 = jnp.zeros((LANES,), jnp.float32)

    @pl.loop(0, N // LANES)
    def _zero(i):
        hist[pl.ds(i * LANES, LANES)] = zeros16

    ones16 = jnp.ones((LANES,), jnp.float32)

    @pl.loop(0, EPW // LANES)
    def _accum(i):
        idx = dstv[pl.ds(i * LANES, LANES)]
        plsc.addupdate_scatter(hist, [idx], ones16)

    tail = EPW - (EPW // LANES) * LANES
    if tail:
        idx = dstv[pl.ds((EPW // LANES) * LANES, LANES)]
        mask = lax.iota(jnp.int32, LANES) < tail
        plsc.addupdate_scatter(hist, [idx], ones16, mask=mask)

    pltpu.sync_copy(hist, out_hbm.at[wid])


@functools.partial(
    pl.kernel,
    out_type=jax.ShapeDtypeStruct((NC * N, H), jnp.float32),
    mesh=_MESH,
    compiler_params=pltpu.CompilerParams(needs_layout_passes=False),
    scratch_types=[
        pltpu.VMEM((NCHUNK, CH), jnp.int32),
        pltpu.VMEM((2, CH), jnp.int32),
        pltpu.VMEM((2, CH), jnp.int32),
        pltpu.VMEM((2, CH, H), jnp.float32),
        pltpu.VMEM_SHARED((N, H), jnp.float32),
        pltpu.SemaphoreType.DMA,
        pltpu.SemaphoreType.DMA,
        pltpu.SemaphoreType.DMA,
        pltpu.SemaphoreType.DMA,
    ],
)
def _sc_mp(zf_hbm, pk_hbm, out_hbm, pkv, srcw, dstw, gbuf, acc,
           g0s, g1s, t0s, t1s):
    c = lax.axis_index("c")
    s = lax.axis_index("s")
    row0 = s * RP8

    di = pltpu.async_copy(zf_hbm.at[pl.ds(c * N + row0, RP8)],
                          acc.at[pl.ds(row0, RP8)], g0s)
    dp = pltpu.async_copy(pk_hbm.at[s], pkv, g1s)

    @pl.when(s == NS - 1)
    def _init_tail():
        pltpu.sync_copy(zf_hbm.at[pl.ds(c * N + TAIL0, TAILN)],
                        acc.at[pl.ds(TAIL0, TAILN)])

    di.wait()
    dp.wait()

    plsc.subcore_barrier()

    offv = jnp.full((LANES,), c * N, jnp.int32)
    lo16 = jnp.full((LANES,), 0xFFFF, jnp.int32)

    _STEPS = [k * LANES for k in range(CH // LANES)]
    if CH % LANES:
        _STEPS.append(CH - LANES)

    def _unpack(j, buf):
        for st in _STEPS:
            sl = pl.ds(st, LANES)
            pv = pkv[j, sl]
            srcw[buf, sl] = (pv & lo16) + offv
            dstw[buf, sl] = lax.shift_right_logical(pv, 16)

    gsem = (g0s, g1s)
    ssem = (t0s, t1s)

    def _gather(buf):
        return pltpu.async_copy(zf_hbm.at[srcw.at[buf]], gbuf.at[buf],
                                gsem[buf])

    def _scatter(buf):
        return pltpu.async_copy(gbuf.at[buf], acc.at[dstw.at[buf]],
                                ssem[buf], add=True)

    U = 20

    @pl.loop(0, NCHUNK // U)
    def _grp(g):
        j0 = g * U
        gd = [None] * U
        sd = [None] * U
        _unpack(j0, 0)
        gd[0] = _gather(0)
        _unpack(j0 + 1, 1)
        gd[1] = _gather(1)
        gd[0].wait()
        sd[0] = _scatter(0)
        for u in range(2, U):
            b = u % 2
            sd[u - 2].wait()
            _unpack(j0 + u, b)
            gd[u] = _gather(b)
            gd[u - 1].wait()
            sd[u - 1] = _scatter(1 - b)
        gd[U - 1].wait()
        sd[U - 1] = _scatter((U - 1) % 2)
        sd[U - 2].wait()
        sd[U - 1].wait()

    plsc.subcore_barrier()
    pltpu.sync_copy(acc.at[pl.ds(row0, RP8)],
                    out_hbm.at[pl.ds(c * N + row0, RP8)])

    @pl.when(s == NS - 1)
    def _wb_tail():
        pltpu.sync_copy(acc.at[pl.ds(TAIL0, TAILN)],
                        out_hbm.at[pl.ds(c * N + TAIL0, TAILN)])


def _tc_prep_body(x_ref, w_ref, degp_ref, z_ref, dinv_ref):
    ones32 = jnp.ones((NC * NS, 1), jnp.float32)
    degp = degp_ref[...].reshape(NC * NS, RB)
    deg = lax.dot_general(degp, ones32,
                          (((0,), (0,)), ((), ())),
                          preferred_element_type=jnp.float32)
    dv = lax.rsqrt(deg + 1.0)
    dinv_ref[...] = dv
    xw = jnp.dot(x_ref[...], w_ref[...], preferred_element_type=jnp.float32)
    z = xw * dv
    z_ref[0] = z[:, :H]
    z_ref[1] = z[:, H:]


_tc_prep = pl.pallas_call(
    _tc_prep_body,
    grid=(NRB,),
    in_specs=[
        pl.BlockSpec((RB, D), lambda i: (i, 0)),
        pl.BlockSpec((D, D), lambda i: (0, 0)),
        pl.BlockSpec((1, NC * NS, RB), lambda i: (i, 0, 0)),
    ],
    out_specs=[
        pl.BlockSpec((2, RB, H), lambda i: (0, i, 0)),
        pl.BlockSpec((RB, 1), lambda i: (i, 0)),
    ],
    out_shape=[
        jax.ShapeDtypeStruct((2, N, H), jnp.float32),
        jax.ShapeDtypeStruct((N, 1), jnp.float32),
    ],
)


def _prelu(t, av):
    return jnp.where(t >= 0, t, av * t)


def _tc_layer_body(acc_ref, dinv_ref, w_ref, b_ref, a_ref, bat_ref,
                   z_ref, pool_ref):
    dv = dinv_ref[...]
    av = a_ref[0, 0]
    h0 = _prelu(acc_ref[0] * dv + b_ref[:, :H], av)
    h1 = _prelu(acc_ref[1] * dv + b_ref[:, H:], av)
    h = jnp.concatenate([h0, h1], axis=1)
    xw = jnp.dot(h, w_ref[...], preferred_element_type=jnp.float32)
    z = xw * dv
    z_ref[0] = z[:, :H]
    z_ref[1] = z[:, H:]

    @pl.when(pl.program_id(0) == 0)
    def _init():
        pool_ref[...] = jnp.zeros((G, D), jnp.float32)

    bat = bat_ref[...].reshape(1, RB)
    gid = lax.broadcasted_iota(jnp.int32, (G, RB), 0)
    ind = jnp.where(gid == bat, 1.0, 0.0)
    pool_ref[...] += jnp.dot(ind, h, preferred_element_type=jnp.float32)


_tc_layer = pl.pallas_call(
    _tc_layer_body,
    grid=(NRB,),
    in_specs=[
        pl.BlockSpec((2, RB, H), lambda i: (0, i, 0)),
        pl.BlockSpec((RB, 1), lambda i: (i, 0)),
        pl.BlockSpec((D, D), lambda i: (0, 0)),
        pl.BlockSpec((1, D), lambda i: (0, 0)),
        pl.BlockSpec(memory_space=pltpu.SMEM),
        pl.BlockSpec((1, 1, RB), lambda i: (i, 0, 0)),
    ],
    out_specs=[
        pl.BlockSpec((2, RB, H), lambda i: (0, i, 0)),
        pl.BlockSpec((G, D), lambda i: (0, 0)),
    ],
    out_shape=[
        jax.ShapeDtypeStruct((2, N, H), jnp.float32),
        jax.ShapeDtypeStruct((G, D), jnp.float32),
    ],
    compiler_params=pltpu.CompilerParams(
        dimension_semantics=("arbitrary",)),
)


def _tc_final_body(acc_ref, dinv_ref, b_ref, a_ref, bat_ref,
                   h3_ref, pool_ref):
    dv = dinv_ref[...]
    av = a_ref[0, 0]
    p0 = _prelu(acc_ref[0] * dv + b_ref[:, :H], av)
    p1 = _prelu(acc_ref[1] * dv + b_ref[:, H:], av)
    h3 = jnp.concatenate([p0, p1], axis=1)
    h3_ref[...] = h3

    @pl.when(pl.program_id(0) == 0)
    def _init():
        pool_ref[...] = jnp.zeros((G, D), jnp.float32)

    bat = bat_ref[...].reshape(1, RB)
    gid = lax.broadcasted_iota(jnp.int32, (G, RB), 0)
    ind = jnp.where(gid == bat, 1.0, 0.0)
    pool_ref[...] += jnp.dot(ind, h3, preferred_element_type=jnp.float32)


_tc_final = pl.pallas_call(
    _tc_final_body,
    grid=(NRB,),
    in_specs=[
        pl.BlockSpec((2, RB, H), lambda i: (0, i, 0)),
        pl.BlockSpec((RB, 1), lambda i: (i, 0)),
        pl.BlockSpec((1, D), lambda i: (0, 0)),
        pl.BlockSpec(memory_space=pltpu.SMEM),
        pl.BlockSpec((1, 1, RB), lambda i: (i, 0, 0)),
    ],
    out_specs=[
        pl.BlockSpec((RB, D), lambda i: (i, 0)),
        pl.BlockSpec((G, D), lambda i: (0, 0)),
    ],
    out_shape=[
        jax.ShapeDtypeStruct((N, D), jnp.float32),
        jax.ShapeDtypeStruct((G, D), jnp.float32),
    ],
    compiler_params=pltpu.CompilerParams(
        dimension_semantics=("arbitrary",)),
)


def kernel(x, edge_index, batch, W0, b0, W1, b1, W2, b2, a):
    src = edge_index[0]
    dst = edge_index[1]

    dstd = jnp.concatenate(
        [dst.reshape(NC * NS, EPW),
         jnp.zeros((NC * NS, EPW_PAD - EPW), jnp.int32)], axis=1)
    pk = ((dst << 16) | src).reshape(NS, NCHUNK, CH)

    batr = batch.reshape(NRB, 1, RB)
    b0r = b0.reshape(1, D)
    b1r = b1.reshape(1, D)
    b2r = b2.reshape(1, D)
    ar = a.reshape(1, 1)

    degp = _sc_degree(dstd)
    degpt = jnp.transpose(degp.reshape(NC * NS, NRB, RB), (1, 0, 2))
    z0, dinv = _tc_prep(x, W0, degpt)

    acc0 = _sc_mp(z0.reshape(2 * N, H), pk)
    z1, p1 = _tc_layer(acc0.reshape(2, N, H), dinv, W1, b0r, ar, batr)

    acc1 = _sc_mp(z1.reshape(2 * N, H), pk)
    z2, p2 = _tc_layer(acc1.reshape(2, N, H), dinv, W2, b1r, ar, batr)

    acc2 = _sc_mp(z2.reshape(2 * N, H), pk)
    h3, p3 = _tc_final(acc2.reshape(2, N, H), dinv, b2r, ar, batr)

    pooled = jnp.concatenate([p1, p2, p3], axis=1)
    return (pooled, h3)


# speedup vs baseline: 19.1817x; 1.0235x over previous
import functools

import jax
import jax.numpy as jnp
from jax import lax
from jax.experimental import pallas as pl
from jax.experimental.pallas import tpu as pltpu
from jax.experimental.pallas import tpu_sc as plsc

N = 10000
E = 160000
D = 256
H = 128
G = 64
NC = 2
NS = 16
LANES = 16

EPT = E // NS
CH = 125
NCHUNK = EPT // CH
RP8 = 624
TAIL0 = NS * RP8
TAILN = N - TAIL0

EPW = E // (NC * NS)
EPW_PAD = ((EPW + 15) // 16) * 16

RB = 2000
NRB = N // RB

_MESH = plsc.VectorSubcoreMesh(core_axis_name="c", subcore_axis_name="s",
                               num_cores=NC, num_subcores=NS)


@functools.partial(
    pl.kernel,
    out_type=jax.ShapeDtypeStruct((NC * NS, N), jnp.float32),
    mesh=_MESH,
    compiler_params=pltpu.CompilerParams(needs_layout_passes=False),
    scratch_types=[
        pltpu.VMEM((EPW_PAD,), jnp.int32),
        pltpu.VMEM((N,), jnp.float32),
    ],
)
def _sc_degree(dst_hbm, out_hbm, dstv, hist):
    wid = lax.axis_index("s") * NC + lax.axis_index("c")
    pltpu.sync_copy(dst_hbm.at[wid], dstv)

    zeros16 = jnp.zeros((LANES,), jnp.float32)

    @pl.loop(0, N // LANES)
    def _zero(i):
        hist[pl.ds(i * LANES, LANES)] = zeros16

    ones16 = jnp.ones((LANES,), jnp.float32)

    @pl.loop(0, EPW // LANES)
    def _accum(i):
        idx = dstv[pl.ds(i * LANES, LANES)]
        plsc.addupdate_scatter(hist, [idx], ones16)

    tail = EPW - (EPW // LANES) * LANES
    if tail:
        idx = dstv[pl.ds((EPW // LANES) * LANES, LANES)]
        mask = lax.iota(jnp.int32, LANES) < tail
        plsc.addupdate_scatter(hist, [idx], ones16, mask=mask)

    pltpu.sync_copy(hist, out_hbm.at[wid])


@functools.partial(
    pl.kernel,
    out_type=jax.ShapeDtypeStruct((NC * N, H), jnp.float32),
    mesh=_MESH,
    compiler_params=pltpu.CompilerParams(needs_layout_passes=False),
    scratch_types=[
        pltpu.VMEM((NCHUNK, CH), jnp.int32),
        pltpu.VMEM((2, CH), jnp.int32),
        pltpu.VMEM((2, CH), jnp.int32),
        pltpu.VMEM((2, CH, H), jnp.float32),
        pltpu.VMEM_SHARED((N, H), jnp.float32),
        pltpu.SemaphoreType.DMA,
        pltpu.SemaphoreType.DMA,
        pltpu.SemaphoreType.DMA,
        pltpu.SemaphoreType.DMA,
    ],
)
def _sc_mp(zf_hbm, pk_hbm, out_hbm, pkv, srcw, dstw, gbuf, acc,
           g0s, g1s, t0s, t1s):
    c = lax.axis_index("c")
    s = lax.axis_index("s")
    row0 = s * RP8

    di = pltpu.async_copy(zf_hbm.at[pl.ds(c * N + row0, RP8)],
                          acc.at[pl.ds(row0, RP8)], g0s)
    dp = pltpu.async_copy(pk_hbm.at[s], pkv, g1s)

    @pl.when(s == NS - 1)
    def _init_tail():
        pltpu.sync_copy(zf_hbm.at[pl.ds(c * N + TAIL0, TAILN)],
                        acc.at[pl.ds(TAIL0, TAILN)])

    di.wait()
    dp.wait()

    plsc.subcore_barrier()

    offv = jnp.full((LANES,), c * N, jnp.int32)
    lo16 = jnp.full((LANES,), 0xFFFF, jnp.int32)

    _STEPS = [k * LANES for k in range(CH // LANES)]
    if CH % LANES:
        _STEPS.append(CH - LANES)

    def _unpack(j, buf):
        for st in _STEPS:
            sl = pl.ds(st, LANES)
            pv = pkv[j, sl]
            srcw[buf, sl] = (pv & lo16) + offv
            dstw[buf, sl] = lax.shift_right_logical(pv, 16)

    gsem = (g0s, g1s)
    ssem = (t0s, t1s)

    def _gather(buf):
        return pltpu.async_copy(zf_hbm.at[srcw.at[buf]], gbuf.at[buf],
                                gsem[buf])

    def _scatter(buf):
        return pltpu.async_copy(gbuf.at[buf], acc.at[dstw.at[buf]],
                                ssem[buf], add=True)

    U = 20

    @pl.loop(0, NCHUNK // U)
    def _grp(g):
        j0 = g * U
        gd = [None] * U
        sd = [None] * U
        _unpack(j0, 0)
        gd[0] = _gather(0)
        _unpack(j0 + 1, 1)
        gd[1] = _gather(1)
        gd[0].wait()
        sd[0] = _scatter(0)
        for u in range(2, U):
            b = u % 2
            sd[u - 2].wait()
            _unpack(j0 + u, b)
            gd[u] = _gather(b)
            gd[u - 1].wait()
            sd[u - 1] = _scatter(1 - b)
        gd[U - 1].wait()
        sd[U - 1] = _scatter((U - 1) % 2)
        sd[U - 2].wait()
        sd[U - 1].wait()

    plsc.subcore_barrier()
    pltpu.sync_copy(acc.at[pl.ds(row0, RP8)],
                    out_hbm.at[pl.ds(c * N + row0, RP8)])

    @pl.when(s == NS - 1)
    def _wb_tail():
        pltpu.sync_copy(acc.at[pl.ds(TAIL0, TAILN)],
                        out_hbm.at[pl.ds(c * N + TAIL0, TAILN)])


def _tc_prep_body(x_ref, w_ref, degp_ref, z_ref, dinv_ref):
    ones32 = jnp.ones((NC * NS, 1), jnp.float32)
    degp = degp_ref[...].reshape(NC * NS, RB)
    deg = lax.dot_general(degp, ones32,
                          (((0,), (0,)), ((), ())),
                          preferred_element_type=jnp.float32)
    dv = lax.rsqrt(deg + 1.0)
    dinv_ref[...] = dv
    xw = jnp.dot(x_ref[...], w_ref[...], preferred_element_type=jnp.float32)
    z = xw * dv
    z_ref[0] = z[:, :H]
    z_ref[1] = z[:, H:]


_tc_prep = pl.pallas_call(
    _tc_prep_body,
    grid=(NRB,),
    in_specs=[
        pl.BlockSpec((RB, D), lambda i: (i, 0)),
        pl.BlockSpec((D, D), lambda i: (0, 0)),
        pl.BlockSpec((1, NC * NS, RB), lambda i: (i, 0, 0)),
    ],
    out_specs=[
        pl.BlockSpec((2, RB, H), lambda i: (0, i, 0)),
        pl.BlockSpec((RB, 1), lambda i: (i, 0)),
    ],
    out_shape=[
        jax.ShapeDtypeStruct((2, N, H), jnp.float32),
        jax.ShapeDtypeStruct((N, 1), jnp.float32),
    ],
)


def _prelu(t, av):
    return jnp.where(t >= 0, t, av * t)


def _tc_layer_body(acc_ref, dinv_ref, w_ref, b_ref, a_ref, bat_ref,
                   z_ref, pool_ref):
    dv = dinv_ref[...]
    av = a_ref[0, 0]
    h0 = _prelu(acc_ref[0] * dv + b_ref[:, :H], av)
    h1 = _prelu(acc_ref[1] * dv + b_ref[:, H:], av)
    h = jnp.concatenate([h0, h1], axis=1)
    xw = jnp.dot(h, w_ref[...], preferred_element_type=jnp.float32)
    z = xw * dv
    z_ref[0] = z[:, :H]
    z_ref[1] = z[:, H:]

    @pl.when(pl.program_id(0) == 0)
    def _init():
        pool_ref[...] = jnp.zeros((G, D), jnp.float32)

    bat = bat_ref[...].reshape(1, RB)
    gid = lax.broadcasted_iota(jnp.int32, (G, RB), 0)
    ind = jnp.where(gid == bat, 1.0, 0.0)
    pool_ref[...] += jnp.dot(ind, h, preferred_element_type=jnp.float32)


_tc_layer = pl.pallas_call(
    _tc_layer_body,
    grid=(NRB,),
    in_specs=[
        pl.BlockSpec((2, RB, H), lambda i: (0, i, 0)),
        pl.BlockSpec((RB, 1), lambda i: (i, 0)),
        pl.BlockSpec((D, D), lambda i: (0, 0)),
        pl.BlockSpec((1, D), lambda i: (0, 0)),
        pl.BlockSpec(memory_space=pltpu.SMEM),
        pl.BlockSpec((1, 1, RB), lambda i: (i, 0, 0)),
    ],
    out_specs=[
        pl.BlockSpec((2, RB, H), lambda i: (0, i, 0)),
        pl.BlockSpec((G, D), lambda i: (0, 0)),
    ],
    out_shape=[
        jax.ShapeDtypeStruct((2, N, H), jnp.float32),
        jax.ShapeDtypeStruct((G, D), jnp.float32),
    ],
    compiler_params=pltpu.CompilerParams(
        dimension_semantics=("arbitrary",)),
)


def _tc_final_body(acc_ref, dinv_ref, b_ref, a_ref, bat_ref,
                   h3_ref, pool_ref):
    dv = dinv_ref[...]
    av = a_ref[0, 0]
    p0 = _prelu(acc_ref[0] * dv + b_ref[:, :H], av)
    p1 = _prelu(acc_ref[1] * dv + b_ref[:, H:], av)
    h3 = jnp.concatenate([p0, p1], axis=1)
    h3_ref[...] = h3

    @pl.when(pl.program_id(0) == 0)
    def _init():
        pool_ref[...] = jnp.zeros((G, D), jnp.float32)

    bat = bat_ref[...].reshape(1, RB)
    gid = lax.broadcasted_iota(jnp.int32, (G, RB), 0)
    ind = jnp.where(gid == bat, 1.0, 0.0)
    pool_ref[...] += jnp.dot(ind, h3, preferred_element_type=jnp.float32)


_tc_final = pl.pallas_call(
    _tc_final_body,
    grid=(NRB,),
    in_specs=[
        pl.BlockSpec((2, RB, H), lambda i: (0, i, 0)),
        pl.BlockSpec((RB, 1), lambda i: (i, 0)),
        pl.BlockSpec((1, D), lambda i: (0, 0)),
        pl.BlockSpec(memory_space=pltpu.SMEM),
        pl.BlockSpec((1, 1, RB), lambda i: (i, 0, 0)),
    ],
    out_specs=[
        pl.BlockSpec((RB, D), lambda i: (i, 0)),
        pl.BlockSpec((G, D), lambda i: (0, 0)),
    ],
    out_shape=[
        jax.ShapeDtypeStruct((N, D), jnp.float32),
        jax.ShapeDtypeStruct((G, D), jnp.float32),
    ],
    compiler_params=pltpu.CompilerParams(
        dimension_semantics=("arbitrary",)),
)


def kernel(x, edge_index, batch, W0, b0, W1, b1, W2, b2, a):
    src = edge_index[0]
    dst = edge_index[1]

    dstd = jnp.concatenate(
        [dst.reshape(NC * NS, EPW),
         jnp.zeros((NC * NS, EPW_PAD - EPW), jnp.int32)], axis=1)
    pk = ((dst << 16) | src).reshape(NS, NCHUNK, CH)

    batr = batch.reshape(NRB, 1, RB)
    b0r = b0.reshape(1, D)
    b1r = b1.reshape(1, D)
    b2r = b2.reshape(1, D)
    ar = a.reshape(1, 1)

    degp = _sc_degree(dstd)
    degpt = jnp.transpose(degp.reshape(NC * NS, NRB, RB), (1, 0, 2))
    z0, dinv = _tc_prep(x, W0, degpt)

    acc0 = _sc_mp(z0.reshape(2 * N, H), pk)
    z1, p1 = _tc_layer(acc0.reshape(2, N, H), dinv, W1, b0r, ar, batr)

    acc1 = _sc_mp(z1.reshape(2 * N, H), pk)
    z2, p2 = _tc_layer(acc1.reshape(2, N, H), dinv, W2, b1r, ar, batr)

    acc2 = _sc_mp(z2.reshape(2 * N, H), pk)
    h3, p3 = _tc_final(acc2.reshape(2, N, H), dinv, b2r, ar, batr)

    pooled = jnp.concatenate([p1, p2, p3], axis=1)
    return (pooled, h3)
